# Initial kernel scaffold; baseline (speedup 1.0000x reference)
import jax
import jax.numpy as jnp
from jax.experimental import pallas as pl


def kernel(x, Wr, Wq, Wk, Wv, Wo):
    raise NotImplementedError("write your pallas kernel here")


# speedup vs baseline: 1.6575x; 1.6575x over previous
import functools

import jax
import jax.numpy as jnp
import numpy as np
from jax.experimental import pallas as pl

_DIM = 768
_HEADS = 12
_HEAD_DIM = _DIM // _HEADS
_E = 8
_TOPK = 2


def _router_body(x_ref, wr_ref, we_ref, lb_ref):
    x = x_ref[...]
    wr = wr_ref[...]
    logits = jax.lax.dot_general(x, wr, (((1,), (1,)), ((), ())),
                                 preferred_element_type=jnp.float32)
    m = jnp.max(logits, axis=-1, keepdims=True)
    ex = jnp.exp(logits - m)
    probs = ex / jnp.sum(ex, axis=-1, keepdims=True)
    T = probs.shape[0]
    E = probs.shape[1]
    iota = jax.lax.broadcasted_iota(jnp.int32, (T, E), 1)
    m1 = jnp.max(probs, axis=-1, keepdims=True)
    i1 = jnp.min(jnp.where(probs == m1, iota, E), axis=-1, keepdims=True)
    sel1 = iota == i1
    masked = jnp.where(sel1, -jnp.inf, probs)
    m2 = jnp.max(masked, axis=-1, keepdims=True)
    i2 = jnp.min(jnp.where(masked == m2, iota, E), axis=-1, keepdims=True)
    sel2 = iota == i2
    denom = m1 + m2 + 1e-6
    we = jnp.where(sel1, m1 / denom, 0.0) + jnp.where(sel2, m2 / denom, 0.0)
    we_ref[...] = we.astype(jnp.float32)
    counts = jnp.sum(sel1.astype(jnp.float32) + sel2.astype(jnp.float32),
                     axis=0)
    p = jnp.sum(probs, axis=0)
    total = jnp.sum(counts)
    frac = counts / (total + 1e-6)
    lb_ref[...] = (jnp.sum(frac * p) * float(E)).reshape(1, 1)


def _router(x_flat, Wr):
    T = x_flat.shape[0]
    we, lb = pl.pallas_call(
        _router_body,
        out_shape=(
            jax.ShapeDtypeStruct((T, _E), jnp.float32),
            jax.ShapeDtypeStruct((1, 1), jnp.float32),
        ),
    )(x_flat, Wr)
    return we, lb[0, 0]


def _moe_body(x_ref, w_ref, we_ref, out_ref, *, chunk):
    e = pl.program_id(0)
    w = w_ref[...].reshape(w_ref.shape[1], w_ref.shape[2])
    we = we_ref[...]
    lane = jax.lax.broadcasted_iota(jnp.int32, we.shape, 1)
    wcol = jnp.sum(jnp.where(lane == e, we, 0.0), axis=1, keepdims=True)
    T = x_ref.shape[0]
    for c in range(T // chunk):
        sl = slice(c * chunk, (c + 1) * chunk)
        acc = jax.lax.dot_general(x_ref[sl, :], w, (((1,), (1,)), ((), ())),
                                  preferred_element_type=jnp.float32)
        contrib = acc * wcol[sl, :]

        @pl.when(e == 0)
        def _():
            out_ref[sl, :] = contrib

        @pl.when(e > 0)
        def _():
            out_ref[sl, :] = out_ref[sl, :] + contrib


def _moe_matmul(x_flat, W, we, chunk=512):
    T, D = x_flat.shape
    E, DO, _ = W.shape
    return pl.pallas_call(
        functools.partial(_moe_body, chunk=chunk),
        grid=(E,),
        in_specs=[
            pl.BlockSpec((T, D), lambda e: (0, 0)),
            pl.BlockSpec((1, DO, D), lambda e: (e, 0, 0)),
            pl.BlockSpec((T, E), lambda e: (0, 0)),
        ],
        out_specs=pl.BlockSpec((T, DO), lambda e: (0, 0)),
        out_shape=jax.ShapeDtypeStruct((T, DO), jnp.float32),
    )(x_flat, W, we)


def _attn_body(qkv_ref, out_ref, *, scale, tq):
    D = _DIM
    Dh = _HEAD_DIM
    base = pl.program_id(0) * tq
    for h in range(_HEADS):
        cs = slice(h * Dh, (h + 1) * Dh)
        q = qkv_ref[pl.ds(base, tq), cs]
        k = qkv_ref[:, D + h * Dh:D + (h + 1) * Dh]
        v = qkv_ref[:, 2 * D + h * Dh:2 * D + (h + 1) * Dh]
        s = jax.lax.dot_general(q, k, (((1,), (1,)), ((), ())),
                                preferred_element_type=jnp.float32) * scale
        m = jnp.max(s, axis=-1, keepdims=True)
        p = jnp.exp(s - m)
        l = jnp.sum(p, axis=-1, keepdims=True)
        o = jax.lax.dot_general(p, v, (((1,), (0,)), ((), ())),
                                preferred_element_type=jnp.float32)
        out_ref[:, cs] = o / l


def _attention(qkv, tq=512):
    T = qkv.shape[0]
    D = _DIM
    scale = 1.0 / np.sqrt(_HEAD_DIM)
    return pl.pallas_call(
        functools.partial(_attn_body, scale=scale, tq=tq),
        grid=(T // tq,),
        in_specs=[
            pl.BlockSpec((T, 3 * D), lambda qi: (0, 0)),
        ],
        out_specs=pl.BlockSpec((tq, D), lambda qi: (qi, 0)),
        out_shape=jax.ShapeDtypeStruct((T, D), jnp.float32),
    )(qkv)


def kernel(x, Wr, Wq, Wk, Wv, Wo):
    B, N, D = x.shape
    x_flat = x.reshape(-1, D)
    we, lb = _router(x_flat, Wr)
    Wqkv = jnp.concatenate([Wq, Wk, Wv], axis=1)
    qkv = _moe_matmul(x_flat, Wqkv, we)
    ctx = _attention(qkv)
    out = _moe_matmul(ctx, Wo, we)
    return out.reshape(B, N, D), lb


# speedup vs baseline: 1.7066x; 1.0296x over previous
import functools

import jax
import jax.numpy as jnp
import numpy as np
from jax.experimental import pallas as pl

_DIM = 768
_HEADS = 12
_HEAD_DIM = _DIM // _HEADS
_E = 8
_TOPK = 2


def _router_body(x_ref, wr_ref, we_ref, lb_ref):
    x = x_ref[...]
    wr = wr_ref[...]
    logits = jax.lax.dot_general(x, wr, (((1,), (1,)), ((), ())),
                                 preferred_element_type=jnp.float32)
    m = jnp.max(logits, axis=-1, keepdims=True)
    ex = jnp.exp(logits - m)
    probs = ex / jnp.sum(ex, axis=-1, keepdims=True)
    T = probs.shape[0]
    E = probs.shape[1]
    iota = jax.lax.broadcasted_iota(jnp.int32, (T, E), 1)
    m1 = jnp.max(probs, axis=-1, keepdims=True)
    i1 = jnp.min(jnp.where(probs == m1, iota, E), axis=-1, keepdims=True)
    sel1 = iota == i1
    masked = jnp.where(sel1, -jnp.inf, probs)
    m2 = jnp.max(masked, axis=-1, keepdims=True)
    i2 = jnp.min(jnp.where(masked == m2, iota, E), axis=-1, keepdims=True)
    sel2 = iota == i2
    denom = m1 + m2 + 1e-6
    we = jnp.where(sel1, m1 / denom, 0.0) + jnp.where(sel2, m2 / denom, 0.0)
    we_ref[...] = we.astype(jnp.float32)
    counts = jnp.sum(sel1.astype(jnp.float32) + sel2.astype(jnp.float32),
                     axis=0)
    p = jnp.sum(probs, axis=0)
    total = jnp.sum(counts)
    frac = counts / (total + 1e-6)
    lb_ref[...] = (jnp.sum(frac * p) * float(E)).reshape(1, 1)


def _router(x_flat, Wr):
    T = x_flat.shape[0]
    we, lb = pl.pallas_call(
        _router_body,
        out_shape=(
            jax.ShapeDtypeStruct((T, _E), jnp.float32),
            jax.ShapeDtypeStruct((1, 1), jnp.float32),
        ),
    )(x_flat, Wr)
    return we, lb[0, 0]


def _moe_body(x_ref, w_ref, we_ref, out_ref, *, chunk):
    e = pl.program_id(0)
    w = w_ref[...].reshape(w_ref.shape[1], w_ref.shape[2])
    we = we_ref[...]
    lane = jax.lax.broadcasted_iota(jnp.int32, we.shape, 1)
    wcol = jnp.sum(jnp.where(lane == e, we, 0.0), axis=1, keepdims=True)
    T = x_ref.shape[0]
    wb = w.astype(jnp.bfloat16)
    for c in range(T // chunk):
        sl = slice(c * chunk, (c + 1) * chunk)
        acc = jax.lax.dot_general(x_ref[sl, :].astype(jnp.bfloat16), wb,
                                  (((1,), (1,)), ((), ())),
                                  preferred_element_type=jnp.float32)
        contrib = acc * wcol[sl, :]

        @pl.when(e == 0)
        def _():
            out_ref[sl, :] = contrib

        @pl.when(e > 0)
        def _():
            out_ref[sl, :] = out_ref[sl, :] + contrib


def _moe_matmul(x_flat, W, we, chunk=512):
    T, D = x_flat.shape
    E, DO, _ = W.shape
    return pl.pallas_call(
        functools.partial(_moe_body, chunk=chunk),
        grid=(E,),
        in_specs=[
            pl.BlockSpec((T, D), lambda e: (0, 0)),
            pl.BlockSpec((1, DO, D), lambda e: (e, 0, 0)),
            pl.BlockSpec((T, E), lambda e: (0, 0)),
        ],
        out_specs=pl.BlockSpec((T, DO), lambda e: (0, 0)),
        out_shape=jax.ShapeDtypeStruct((T, DO), jnp.float32),
    )(x_flat, W, we)


def _attn_body(qkv_ref, out_ref, *, scale, tq):
    D = _DIM
    Dh = _HEAD_DIM
    base = pl.program_id(0) * tq
    for h in range(_HEADS):
        cs = slice(h * Dh, (h + 1) * Dh)
        q = qkv_ref[pl.ds(base, tq), cs].astype(jnp.bfloat16)
        k = qkv_ref[:, D + h * Dh:D + (h + 1) * Dh].astype(jnp.bfloat16)
        v = qkv_ref[:, 2 * D + h * Dh:2 * D + (h + 1) * Dh].astype(jnp.bfloat16)
        s = jax.lax.dot_general(q, k, (((1,), (1,)), ((), ())),
                                preferred_element_type=jnp.float32) * scale
        m = jnp.max(s, axis=-1, keepdims=True)
        p = jnp.exp(s - m)
        l = jnp.sum(p, axis=-1, keepdims=True)
        o = jax.lax.dot_general(p.astype(jnp.bfloat16), v,
                                (((1,), (0,)), ((), ())),
                                preferred_element_type=jnp.float32)
        out_ref[:, cs] = o / l


def _attention(qkv, tq=512):
    T = qkv.shape[0]
    D = _DIM
    scale = 1.0 / np.sqrt(_HEAD_DIM)
    return pl.pallas_call(
        functools.partial(_attn_body, scale=scale, tq=tq),
        grid=(T // tq,),
        in_specs=[
            pl.BlockSpec((T, 3 * D), lambda qi: (0, 0)),
        ],
        out_specs=pl.BlockSpec((tq, D), lambda qi: (qi, 0)),
        out_shape=jax.ShapeDtypeStruct((T, D), jnp.float32),
    )(qkv)


def kernel(x, Wr, Wq, Wk, Wv, Wo):
    B, N, D = x.shape
    x_flat = x.reshape(-1, D)
    we, lb = _router(x_flat, Wr)
    Wqkv = jnp.concatenate([Wq, Wk, Wv], axis=1)
    qkv = _moe_matmul(x_flat, Wqkv, we)
    ctx = _attention(qkv)
    out = _moe_matmul(ctx, Wo, we)
    return out.reshape(B, N, D), lb
